# baseline (speedup 1.0000x reference)
import jax
import jax.numpy as jnp
from jax import lax
from jax.experimental import pallas as pl
from jax.experimental.pallas import tpu as pltpu
from jax.experimental.pallas import tpu_sc as plsc
from jax._src.pallas import mpmd as _mpmd

B, S, H, I = 16384, 50, 64, 64
P = B * S // 2
NW = 32
BPW = B // NW
CH = 128
NCH = BPW // CH


def _mesh():
    return plsc.VectorSubcoreMesh(core_axis_name="c", subcore_axis_name="s")


def _worker_base():
    nc = plsc.get_sparse_core_info().num_cores
    return (lax.axis_index("s") * nc + lax.axis_index("c")) * BPW


def _gather_body(tblh, tblc, gidx, sidx, gh, gc, sh, sc, tblh_out, tblc_out,
                 gidx_v, sidx_v, bh_v, bc_v, bsh_v, bsc_v, sem):
    del tblh_out, tblc_out
    wid = _worker_base() // BPW
    base = wid * BPW
    pltpu.sync_copy(gidx.at[wid], gidx_v)
    pltpu.sync_copy(sidx.at[wid], sidx_v)
    for j in range(NCH):
        c1 = pltpu.async_copy(tblh.at[gidx_v.at[j]], bh_v, sem)
        c2 = pltpu.async_copy(tblc.at[gidx_v.at[j]], bc_v, sem)
        c3 = pltpu.async_copy(tblh.at[sidx_v.at[j]], bsh_v, sem)
        c4 = pltpu.async_copy(tblc.at[sidx_v.at[j]], bsc_v, sem)
        c1.wait(); c2.wait(); c3.wait(); c4.wait()
        off = base + j * CH
        pltpu.sync_copy(bh_v, gh.at[pl.ds(off, CH)])
        pltpu.sync_copy(bc_v, gc.at[pl.ds(off, CH)])
        pltpu.sync_copy(bsh_v, sh.at[pl.ds(off, CH)])
        pltpu.sync_copy(bsc_v, sc.at[pl.ds(off, CH)])


def _sc_gather(tblh, tblc, gidx3, sidx3):
    k = _mpmd._mpmd_map(
        [(_mesh(), _gather_body)],
        [jax.ShapeDtypeStruct((B, 128), jnp.float32) for _ in range(4)]
        + [jax.ShapeDtypeStruct((P, 128), jnp.float32),
           jax.ShapeDtypeStruct((P, 128), jnp.float32)],
        input_output_aliases={0: 4, 1: 5},
        scratch_types=[
            pltpu.VMEM((NCH, CH), jnp.int32),
            pltpu.VMEM((NCH, CH), jnp.int32),
            pltpu.VMEM((CH, 128), jnp.float32),
            pltpu.VMEM((CH, 128), jnp.float32),
            pltpu.VMEM((CH, 128), jnp.float32),
            pltpu.VMEM((CH, 128), jnp.float32),
            pltpu.SemaphoreType.DMA,
        ],
    )
    return k(tblh, tblc, gidx3, sidx3)


def _scatter_body(tblh_in, tblc_in, sidx, ph, pc, outh, outc,
                  sidx_v, pv_h, pv_c, sem):
    del tblh_in, tblc_in
    base = _worker_base()
    wid = base // BPW
    pltpu.sync_copy(sidx.at[wid], sidx_v)
    for j in range(NCH):
        off = base + j * CH
        pltpu.sync_copy(ph.at[pl.ds(off, CH)], pv_h)
        pltpu.sync_copy(pc.at[pl.ds(off, CH)], pv_c)
        c1 = pltpu.async_copy(pv_h, outh.at[sidx_v.at[j]], sem)
        c2 = pltpu.async_copy(pv_c, outc.at[sidx_v.at[j]], sem)
        c1.wait(); c2.wait()


def _sc_scatter(tblh, tblc, sidx3, ph, pc):
    k = _mpmd._mpmd_map(
        [(_mesh(), _scatter_body)],
        [jax.ShapeDtypeStruct((P, 128), jnp.float32),
         jax.ShapeDtypeStruct((P, 128), jnp.float32)],
        input_output_aliases={0: 0, 1: 1},
        scratch_types=[
            pltpu.VMEM((NCH, CH), jnp.int32),
            pltpu.VMEM((CH, 128), jnp.float32),
            pltpu.VMEM((CH, 128), jnp.float32),
            pltpu.SemaphoreType.DMA,
        ],
    )
    return k(tblh, tblc, sidx3, ph, pc)


def _lstm_body(pos_ref, sub_ref, gh_ref, gc_ref, sh_ref, sc_ref, w_ref, b_ref,
               hout_ref, cout_ref, ph_ref, pc_ref):
    pos = pos_ref[...]
    podd = (pos % 2) == 1
    gh = gh_ref[...]
    gc = gc_ref[...]
    h = jnp.where(podd, gh[:, H:], gh[:, :H])
    c = jnp.where(podd, gc[:, H:], gc[:, :H])
    x = jnp.concatenate([sub_ref[...], h], axis=1)
    gates = jnp.dot(x, w_ref[...], preferred_element_type=jnp.float32)
    gates = gates + b_ref[...]
    i_g = jax.nn.sigmoid(gates[:, 0:H])
    f_g = jax.nn.sigmoid(gates[:, H:2 * H])
    g_g = jnp.tanh(gates[:, 2 * H:3 * H])
    o_g = jax.nn.sigmoid(gates[:, 3 * H:4 * H])
    c_new = f_g * c + i_g * g_g
    h_new = o_g * jnp.tanh(c_new)
    hout_ref[...] = h_new
    cout_ref[...] = c_new
    lane = lax.broadcasted_iota(jnp.int32, (pos.shape[0], 128), 1)
    in_q_half = (lane // H) == (1 - pos % 2)
    duph = jnp.concatenate([h_new, h_new], axis=1)
    dupc = jnp.concatenate([c_new, c_new], axis=1)
    ph_ref[...] = jnp.where(in_q_half, duph, sh_ref[...])
    pc_ref[...] = jnp.where(in_q_half, dupc, sc_ref[...])


def _tc_lstm(pos2d, subword, gh, gc, shp, scp, w, bias):
    BB = 2048
    spec64 = pl.BlockSpec((BB, I), lambda i: (i, 0))
    spec128 = pl.BlockSpec((BB, 128), lambda i: (i, 0))
    return pl.pallas_call(
        _lstm_body,
        grid=(B // BB,),
        in_specs=[
            pl.BlockSpec((BB, 1), lambda i: (i, 0)),
            spec64, spec128, spec128, spec128, spec128,
            pl.BlockSpec((I + H, 4 * H), lambda i: (0, 0)),
            pl.BlockSpec((1, 4 * H), lambda i: (0, 0)),
        ],
        out_specs=[
            pl.BlockSpec((BB, H), lambda i: (i, 0)),
            pl.BlockSpec((BB, H), lambda i: (i, 0)),
            spec128, spec128,
        ],
        out_shape=[
            jax.ShapeDtypeStruct((B, H), jnp.float32),
            jax.ShapeDtypeStruct((B, H), jnp.float32),
            jax.ShapeDtypeStruct((B, 128), jnp.float32),
            jax.ShapeDtypeStruct((B, 128), jnp.float32),
        ],
    )(pos2d, subword, gh, gc, shp, scp, w, bias)


def kernel(subword, stack_hidden, stack_cell, idx, pos,
           weight_ih, weight_hh, bias_ih, bias_hh):
    del idx
    w = jnp.concatenate([weight_ih.T, weight_hh.T], axis=0)
    bias = (bias_ih + bias_hh).reshape(1, 4 * H)
    tblh = stack_hidden.reshape(P, 128)
    tblc = stack_cell.reshape(P, 128)
    rows = jnp.arange(B, dtype=jnp.int32) * S
    gpair = (rows + pos) // 2
    spair = (rows + pos + 1) // 2
    gidx3 = gpair.reshape(NW, NCH, CH)
    sidx3 = spair.reshape(NW, NCH, CH)
    gh, gc, shp, scp, tblh2, tblc2 = _sc_gather(tblh, tblc, gidx3, sidx3)
    h_new, c_new, ph, pc = _tc_lstm(
        pos.reshape(B, 1), subword, gh, gc, shp, scp, w, bias)
    sh_new, sc_new = _sc_scatter(tblh2, tblc2, sidx3, ph, pc)
    return (h_new, c_new,
            sh_new.reshape(B, S, H), sc_new.reshape(B, S, H))


# speedup vs baseline: 7.0682x; 7.0682x over previous
import jax
import jax.numpy as jnp
from jax import lax
from jax.experimental import pallas as pl

B, S, H, I = 16384, 50, 64, 64
BB = 512


def _body(pos_ref, sub_ref, sh_ref, sc_ref, w_ref, b_ref,
          hout_ref, cout_ref, shout_ref, scout_ref):
    pos = pos_ref[...]
    x3h = sh_ref[...]
    x3c = sc_ref[...]
    s_iota = lax.broadcasted_iota(jnp.int32, (S, 1, 1), 0)
    pm = pos[None, :, :]
    maskg = s_iota == pm
    h = jnp.sum(jnp.where(maskg, x3h, 0.0), axis=0)
    c = jnp.sum(jnp.where(maskg, x3c, 0.0), axis=0)
    x = jnp.concatenate([sub_ref[...], h], axis=0)
    gates = jnp.dot(w_ref[...], x, preferred_element_type=jnp.float32)
    gates = gates + b_ref[...]
    i_g = jax.nn.sigmoid(gates[0:H])
    f_g = jax.nn.sigmoid(gates[H:2 * H])
    g_g = jnp.tanh(gates[2 * H:3 * H])
    o_g = jax.nn.sigmoid(gates[3 * H:4 * H])
    c_new = f_g * c + i_g * g_g
    h_new = o_g * jnp.tanh(c_new)
    hout_ref[...] = h_new
    cout_ref[...] = c_new
    masks = s_iota == pm + 1
    shout_ref[...] = jnp.where(masks, h_new[None], x3h)
    scout_ref[...] = jnp.where(masks, c_new[None], x3c)


def kernel(subword, stack_hidden, stack_cell, idx, pos,
           weight_ih, weight_hh, bias_ih, bias_hh):
    del idx
    subt = subword.T
    sht = jnp.transpose(stack_hidden, (1, 2, 0))
    sct = jnp.transpose(stack_cell, (1, 2, 0))
    w = jnp.concatenate([weight_ih, weight_hh], axis=1)
    bias = (bias_ih + bias_hh).reshape(4 * H, 1)
    pos2d = pos.reshape(1, B)
    grid = (B // BB,)
    out = pl.pallas_call(
        _body,
        grid=grid,
        in_specs=[
            pl.BlockSpec((1, BB), lambda i: (0, i)),
            pl.BlockSpec((I, BB), lambda i: (0, i)),
            pl.BlockSpec((S, H, BB), lambda i: (0, 0, i)),
            pl.BlockSpec((S, H, BB), lambda i: (0, 0, i)),
            pl.BlockSpec((4 * H, I + H), lambda i: (0, 0)),
            pl.BlockSpec((4 * H, 1), lambda i: (0, 0)),
        ],
        out_specs=[
            pl.BlockSpec((H, BB), lambda i: (0, i)),
            pl.BlockSpec((H, BB), lambda i: (0, i)),
            pl.BlockSpec((S, H, BB), lambda i: (0, 0, i)),
            pl.BlockSpec((S, H, BB), lambda i: (0, 0, i)),
        ],
        out_shape=[
            jax.ShapeDtypeStruct((H, B), jnp.float32),
            jax.ShapeDtypeStruct((H, B), jnp.float32),
            jax.ShapeDtypeStruct((S, H, B), jnp.float32),
            jax.ShapeDtypeStruct((S, H, B), jnp.float32),
        ],
    )(pos2d, subt, sht, sct, w, bias)
    h_t, c_t, sh_t, sc_t = out
    return (h_t.T, c_t.T,
            jnp.transpose(sh_t, (2, 0, 1)),
            jnp.transpose(sc_t, (2, 0, 1)))
